# baseline (speedup 1.0000x reference)
import functools

import jax
import jax.numpy as jnp
from jax import lax
from jax.experimental import pallas as pl
from jax.experimental.pallas import tpu as pltpu
from jax.experimental.pallas import tpu_sc as plsc

N = 10000
E = 320000
D = 128
NCLS = 40
NPAD = 10240
NC, NS = 2, 16
KD = 2000
KCMP = 80
NBUF = 4
ROWS_PER_TILE = NPAD // NS

_MESH = dict(core_axis_name="c", subcore_axis_name="s")


HROWS = NPAD // D


def _deg_body(ei_hbm, zeros_hbm, out_hbm, hs, hd, src_v, dst_v,
              iota_s, iota_d, acc_sh):
    c = lax.axis_index("c")
    s = lax.axis_index("s")
    pltpu.sync_copy(zeros_hbm.at[pl.ds(0, HROWS)], hs)
    pltpu.sync_copy(zeros_hbm.at[pl.ds(0, HROWS)], hd)

    @pl.when(s < 2 * HROWS // 16)
    def _():
        pltpu.sync_copy(zeros_hbm.at[pl.ds(0, 16)],
                        acc_sh.at[pl.ds(s * 16, 16)])
    for k in range(HROWS // 16):
        base16 = lax.iota(jnp.int32, 16) + (16 * k)
        iota_s[pl.ds(16 * k, 16)] = base16
        iota_d[pl.ds(16 * k, 16)] = base16 + HROWS
    plsc.subcore_barrier()

    ept = E // (NC * NS)
    base = (c * NS + s) * ept

    def chunk(j, carry):
        off = base + j * KD
        pltpu.sync_copy(ei_hbm.at[pl.ds(off, KD)], src_v)
        pltpu.sync_copy(ei_hbm.at[pl.ds(E + off, KD)], dst_v)

        def inner(i, carry2):
            v = src_v[pl.ds(i * 16, 16)]
            cnt, last = plsc.scan_count(v)
            plsc.addupdate_scatter(
                hs, [v >> 7, v & 127], cnt.astype(jnp.float32), mask=last)
            w = dst_v[pl.ds(i * 16, 16)]
            cnt2, last2 = plsc.scan_count(w)
            plsc.addupdate_scatter(
                hd, [w >> 7, w & 127], cnt2.astype(jnp.float32), mask=last2)
            return carry2

        lax.fori_loop(0, KD // 16, inner, 0)
        return carry

    lax.fori_loop(0, ept // KD, chunk, 0)

    pltpu.sync_copy(hs, acc_sh.at[iota_s], add=True)
    pltpu.sync_copy(hd, acc_sh.at[iota_d], add=True)
    plsc.subcore_barrier()

    @pl.when(s < 2 * HROWS // 16)
    def _():
        pltpu.sync_copy(acc_sh.at[pl.ds(s * 16, 16)],
                        out_hbm.at[c, pl.ds(s * 16, 16)])


_deg_kernel = functools.partial(
    pl.kernel,
    out_type=jax.ShapeDtypeStruct((NC, 2 * HROWS, D), jnp.float32),
    mesh=plsc.VectorSubcoreMesh(**_MESH),
    scratch_types=[
        pltpu.VMEM((HROWS, D), jnp.float32),
        pltpu.VMEM((HROWS, D), jnp.float32),
        pltpu.VMEM((KD,), jnp.int32),
        pltpu.VMEM((KD,), jnp.int32),
        pltpu.VMEM((HROWS,), jnp.int32),
        pltpu.VMEM((HROWS,), jnp.int32),
        pltpu.VMEM_SHARED((2 * HROWS, D), jnp.float32),
    ],
    compiler_params=pltpu.CompilerParams(needs_layout_passes=False),
)(_deg_body)


def _mp_body(h_hbm, ei_hbm, zeros_hbm, out_hbm, acc_sh,
             src0, src1, src2, src3, dst0, dst1, dst2, dst3,
             rows0, rows1, rows2, rows3,
             gs0, gs1, gs2, gs3, ss0, ss1, ss2, ss3):
    c = lax.axis_index("c")
    s = lax.axis_index("s")
    r0 = s * ROWS_PER_TILE
    pltpu.sync_copy(zeros_hbm.at[pl.ds(r0, ROWS_PER_TILE)],
                    acc_sh.at[pl.ds(r0, ROWS_PER_TILE)])
    plsc.subcore_barrier()

    ept = E // (NC * NS)
    base = (c * NS + s) * ept
    nchunks = ept // KCMP
    srcs = [src0, src1, src2, src3]
    dsts = [dst0, dst1, dst2, dst3]
    rows = [rows0, rows1, rows2, rows3]
    gsems = [gs0, gs1, gs2, gs3]
    ssems = [ss0, ss1, ss2, ss3]

    def start_gather(j, b):
        pltpu.sync_copy(ei_hbm.at[pl.ds(base + j * KCMP, KCMP)], srcs[b])
        pltpu.async_copy(h_hbm.at[srcs[b]], rows[b], gsems[b])

    def start_scatter(j, b):
        pltpu.make_async_copy(h_hbm.at[srcs[b]], rows[b], gsems[b]).wait()
        pltpu.sync_copy(ei_hbm.at[pl.ds(E + base + j * KCMP, KCMP)], dsts[b])
        pltpu.async_copy(rows[b], acc_sh.at[dsts[b]], ssems[b], add=True)

    def wait_scatter(b):
        pltpu.make_async_copy(rows[b], acc_sh.at[dsts[b]], ssems[b]).wait()

    start_gather(0, 0)
    start_gather(1, 1)
    start_scatter(0, 0)
    start_gather(2, 2)
    start_scatter(1, 1)
    start_gather(3, 3)
    start_scatter(2, 2)
    wait_scatter(0)
    start_gather(4, 0)
    start_scatter(3, 3)
    wait_scatter(1)
    start_gather(5, 1)

    def step(g, carry):
        for b in range(NBUF):
            j = 4 + NBUF * g + b
            start_scatter(j, b)
            wait_scatter((b + 2) % NBUF)
            start_gather(j + 2, (b + 2) % NBUF)
        return carry

    ngroups = (nchunks - 4 - 5) // NBUF
    lax.fori_loop(0, ngroups, step, 0)
    jt = 4 + NBUF * ngroups
    start_scatter(jt, 0)
    wait_scatter(2)
    start_gather(jt + 2, 2)
    start_scatter(jt + 1, 1)
    wait_scatter(3)
    start_gather(jt + 3, 3)
    start_scatter(jt + 2, 2)
    wait_scatter(0)
    start_gather(jt + 4, 0)
    start_scatter(jt + 3, 3)
    wait_scatter(1)
    start_scatter(jt + 4, 0)
    wait_scatter(2)
    wait_scatter(3)
    wait_scatter(0)

    plsc.subcore_barrier()
    pltpu.sync_copy(acc_sh.at[pl.ds(r0, ROWS_PER_TILE)],
                    out_hbm.at[c, pl.ds(r0, ROWS_PER_TILE)])


_mp = functools.partial(
    pl.kernel,
    out_type=jax.ShapeDtypeStruct((NC, NPAD, D), jnp.float32),
    mesh=plsc.VectorSubcoreMesh(**_MESH),
    scratch_types=(
        [pltpu.VMEM_SHARED((NPAD, D), jnp.float32)]
        + [pltpu.VMEM((KCMP,), jnp.int32) for _ in range(8)]
        + [pltpu.VMEM((KCMP, D), jnp.float32) for _ in range(4)]
        + [pltpu.SemaphoreType.DMA for _ in range(8)]
    ),
)(_mp_body)


def _mm_body(x_ref, w_ref, o_ref):
    o_ref[...] = jnp.dot(x_ref[...], w_ref[...],
                         preferred_element_type=jnp.float32)


def _prep_body(xw_ref, degp_ref, o_h, o_ns, o_nd):
    degp_full = degp_ref[...]
    degp = degp_full[0] + degp_full[1]
    deg_out = degp[:HROWS].reshape(NPAD)[:N]
    deg_in = degp[HROWS:].reshape(NPAD)[:N]
    ns = jnp.where(deg_out > 0, lax.rsqrt(deg_out), 0.0)[:, None]
    nd = jnp.where(deg_in > 0, lax.rsqrt(deg_in), 0.0)[:, None]
    o_h[...] = xw_ref[...] * ns
    o_ns[...] = ns
    o_nd[...] = nd


def _mid_body(aggp_ref, nd_ref, b_ref, g_ref, be_ref, ns_ref, w_ref, o_ref):
    aggp = aggp_ref[...]
    agg = aggp[0, :N, :] + aggp[1, :N, :]
    t = agg * nd_ref[...] + b_ref[...]
    mu = jnp.mean(t, axis=0, keepdims=True)
    var = jnp.mean((t - mu) * (t - mu), axis=0, keepdims=True)
    t = (t - mu) * lax.rsqrt(var + 1e-5) * g_ref[...] + be_ref[...]
    t = jnp.maximum(t, 0.0)
    o_ref[...] = jnp.dot(t * ns_ref[...], w_ref[...],
                         preferred_element_type=jnp.float32)


def _fin_body(aggp_ref, nd_ref, b_ref, o_ref):
    aggp = aggp_ref[...]
    agg = aggp[0, :N, :NCLS] + aggp[1, :N, :NCLS]
    o_ref[...] = agg * nd_ref[...] + b_ref[...]


def _tc(body, out_shape, *args):
    return pl.pallas_call(body, out_shape=out_shape)(*args)


def kernel(x, edge_index, W0, b0, g0, be0, W1, b1, g1, be1, W2, b2):
    f32 = jnp.float32
    ei = edge_index.reshape(2 * E)
    zeros_d = jnp.zeros((NPAD, D), f32)
    W2p = jnp.pad(W2, ((0, 0), (0, D - NCLS)))

    degp = _deg_kernel(ei, zeros_d)
    xw = _tc(_mm_body, jax.ShapeDtypeStruct((N, D), f32), x, W0)
    h0, ns, nd = _tc(
        _prep_body,
        (jax.ShapeDtypeStruct((N, D), f32),
         jax.ShapeDtypeStruct((N, 1), f32),
         jax.ShapeDtypeStruct((N, 1), f32)),
        xw, degp)

    aggp0 = _mp(h0, ei, zeros_d)
    h1 = _tc(_mid_body, jax.ShapeDtypeStruct((N, D), f32),
             aggp0, nd, b0.reshape(1, D), g0.reshape(1, D),
             be0.reshape(1, D), ns, W1)

    aggp1 = _mp(h1, ei, zeros_d)
    h2 = _tc(_mid_body, jax.ShapeDtypeStruct((N, D), f32),
             aggp1, nd, b1.reshape(1, D), g1.reshape(1, D),
             be1.reshape(1, D), ns, W2p)

    aggp2 = _mp(h2, ei, zeros_d)
    out = _tc(_fin_body, jax.ShapeDtypeStruct((N, NCLS), f32),
              aggp2, nd, b2.reshape(1, NCLS))
    return out


# speedup vs baseline: 1.0738x; 1.0738x over previous
import functools

import jax
import jax.numpy as jnp
from jax import lax
from jax.experimental import pallas as pl
from jax.experimental.pallas import tpu as pltpu
from jax.experimental.pallas import tpu_sc as plsc

N = 10000
E = 320000
D = 128
NCLS = 40
NPAD = 10240
NC, NS = 2, 16
KD = 2000
KCMP = 80
NBUF = 4
ROWS_PER_TILE = NPAD // NS

_MESH = dict(core_axis_name="c", subcore_axis_name="s")


HROWS = NPAD // D


def _deg_body(ei_hbm, zeros_hbm, out_hbm, hs, hd, src_v, dst_v,
              iota_s, iota_d, acc_sh):
    c = lax.axis_index("c")
    s = lax.axis_index("s")
    pltpu.sync_copy(zeros_hbm.at[pl.ds(0, HROWS)], hs)
    pltpu.sync_copy(zeros_hbm.at[pl.ds(0, HROWS)], hd)

    @pl.when(s < 2 * HROWS // 16)
    def _():
        pltpu.sync_copy(zeros_hbm.at[pl.ds(0, 16)],
                        acc_sh.at[pl.ds(s * 16, 16)])
    for k in range(HROWS // 16):
        base16 = lax.iota(jnp.int32, 16) + (16 * k)
        iota_s[pl.ds(16 * k, 16)] = base16
        iota_d[pl.ds(16 * k, 16)] = base16 + HROWS
    plsc.subcore_barrier()

    ept = E // (NC * NS)
    base = (c * NS + s) * ept

    def chunk(j, carry):
        off = base + j * KD
        pltpu.sync_copy(ei_hbm.at[pl.ds(off, KD)], src_v)
        pltpu.sync_copy(ei_hbm.at[pl.ds(E + off, KD)], dst_v)

        def inner(i, carry2):
            v = src_v[pl.ds(i * 16, 16)]
            cnt, last = plsc.scan_count(v)
            plsc.addupdate_scatter(
                hs, [v >> 7, v & 127], cnt.astype(jnp.float32), mask=last)
            w = dst_v[pl.ds(i * 16, 16)]
            cnt2, last2 = plsc.scan_count(w)
            plsc.addupdate_scatter(
                hd, [w >> 7, w & 127], cnt2.astype(jnp.float32), mask=last2)
            return carry2

        lax.fori_loop(0, KD // 16, inner, 0)
        return carry

    lax.fori_loop(0, ept // KD, chunk, 0)

    pltpu.sync_copy(hs, acc_sh.at[iota_s], add=True)
    pltpu.sync_copy(hd, acc_sh.at[iota_d], add=True)
    plsc.subcore_barrier()

    @pl.when(s < 2 * HROWS // 16)
    def _():
        pltpu.sync_copy(acc_sh.at[pl.ds(s * 16, 16)],
                        out_hbm.at[c, pl.ds(s * 16, 16)])


_deg_kernel = functools.partial(
    pl.kernel,
    out_type=jax.ShapeDtypeStruct((NC, 2 * HROWS, D), jnp.float32),
    mesh=plsc.VectorSubcoreMesh(**_MESH),
    scratch_types=[
        pltpu.VMEM((HROWS, D), jnp.float32),
        pltpu.VMEM((HROWS, D), jnp.float32),
        pltpu.VMEM((KD,), jnp.int32),
        pltpu.VMEM((KD,), jnp.int32),
        pltpu.VMEM((HROWS,), jnp.int32),
        pltpu.VMEM((HROWS,), jnp.int32),
        pltpu.VMEM_SHARED((2 * HROWS, D), jnp.float32),
    ],
    compiler_params=pltpu.CompilerParams(needs_layout_passes=False),
)(_deg_body)


def _mp_body(h_hbm, ei_hbm, zeros_hbm, out_hbm, acc_sh,
             src0, src1, src2, src3, dst0, dst1, dst2, dst3,
             rows0, rows1, rows2, rows3,
             gs0, gs1, gs2, gs3, ss0, ss1, ss2, ss3):
    c = lax.axis_index("c")
    s = lax.axis_index("s")
    r0 = s * ROWS_PER_TILE
    pltpu.sync_copy(zeros_hbm.at[pl.ds(r0, ROWS_PER_TILE)],
                    acc_sh.at[pl.ds(r0, ROWS_PER_TILE)])
    plsc.subcore_barrier()

    ept = E // (NC * NS)
    base = (c * NS + s) * ept
    nchunks = ept // KCMP
    srcs = [src0, src1, src2, src3]
    dsts = [dst0, dst1, dst2, dst3]
    rows = [rows0, rows1, rows2, rows3]
    gsems = [gs0, gs1, gs2, gs3]
    ssems = [ss0, ss1, ss2, ss3]

    def start_gather(j, b):
        pltpu.sync_copy(ei_hbm.at[pl.ds(base + j * KCMP, KCMP)], srcs[b])
        pltpu.async_copy(h_hbm.at[srcs[b]], rows[b], gsems[b])

    def start_scatter(j, b):
        pltpu.make_async_copy(h_hbm.at[srcs[b]], rows[b], gsems[b]).wait()
        pltpu.sync_copy(ei_hbm.at[pl.ds(E + base + j * KCMP, KCMP)], dsts[b])
        pltpu.async_copy(rows[b], acc_sh.at[dsts[b]], ssems[b], add=True)

    def wait_scatter(b):
        pltpu.make_async_copy(rows[b], acc_sh.at[dsts[b]], ssems[b]).wait()

    start_gather(0, 0)
    start_gather(1, 1)
    start_gather(2, 2)
    start_scatter(0, 0)
    start_gather(3, 3)
    for j in range(1, 4):
        start_scatter(j, j)
        wait_scatter(j - 1)
        start_gather(j + 3, (j + 3) % NBUF)

    def step(g, carry):
        for b in range(NBUF):
            j = 4 + NBUF * g + b
            start_scatter(j, b)
            wait_scatter((b + 3) % NBUF)
            start_gather(j + 3, (b + 3) % NBUF)
        return carry

    ngroups = (nchunks - 4 - 5) // NBUF
    lax.fori_loop(0, ngroups, step, 0)
    jt = 4 + NBUF * ngroups
    start_scatter(jt, 0)
    wait_scatter(3)
    start_gather(jt + 3, 3)
    start_scatter(jt + 1, 1)
    wait_scatter(0)
    start_gather(jt + 4, 0)
    start_scatter(jt + 2, 2)
    start_scatter(jt + 3, 3)
    start_scatter(jt + 4, 0)
    wait_scatter(1)
    wait_scatter(2)
    wait_scatter(3)
    wait_scatter(0)

    plsc.subcore_barrier()
    pltpu.sync_copy(acc_sh.at[pl.ds(r0, ROWS_PER_TILE)],
                    out_hbm.at[c, pl.ds(r0, ROWS_PER_TILE)])


_mp = functools.partial(
    pl.kernel,
    out_type=jax.ShapeDtypeStruct((NC, NPAD, D), jnp.float32),
    mesh=plsc.VectorSubcoreMesh(**_MESH),
    scratch_types=(
        [pltpu.VMEM_SHARED((NPAD, D), jnp.float32)]
        + [pltpu.VMEM((KCMP,), jnp.int32) for _ in range(8)]
        + [pltpu.VMEM((KCMP, D), jnp.float32) for _ in range(4)]
        + [pltpu.SemaphoreType.DMA for _ in range(8)]
    ),
)(_mp_body)


def _mm_body(x_ref, w_ref, o_ref):
    o_ref[...] = jnp.dot(x_ref[...], w_ref[...],
                         preferred_element_type=jnp.float32)


def _prep_body(xw_ref, degp_ref, o_h, o_ns, o_nd):
    degp_full = degp_ref[...]
    degp = degp_full[0] + degp_full[1]
    deg_out = degp[:HROWS].reshape(NPAD)[:N]
    deg_in = degp[HROWS:].reshape(NPAD)[:N]
    ns = jnp.where(deg_out > 0, lax.rsqrt(deg_out), 0.0)[:, None]
    nd = jnp.where(deg_in > 0, lax.rsqrt(deg_in), 0.0)[:, None]
    o_h[...] = xw_ref[...] * ns
    o_ns[...] = ns
    o_nd[...] = nd


def _mid_body(aggp_ref, nd_ref, b_ref, g_ref, be_ref, ns_ref, w_ref, o_ref):
    aggp = aggp_ref[...]
    agg = aggp[0, :N, :] + aggp[1, :N, :]
    t = agg * nd_ref[...] + b_ref[...]
    mu = jnp.mean(t, axis=0, keepdims=True)
    var = jnp.mean((t - mu) * (t - mu), axis=0, keepdims=True)
    t = (t - mu) * lax.rsqrt(var + 1e-5) * g_ref[...] + be_ref[...]
    t = jnp.maximum(t, 0.0)
    o_ref[...] = jnp.dot(t * ns_ref[...], w_ref[...],
                         preferred_element_type=jnp.float32)


def _fin_body(aggp_ref, nd_ref, b_ref, o_ref):
    aggp = aggp_ref[...]
    agg = aggp[0, :N, :NCLS] + aggp[1, :N, :NCLS]
    o_ref[...] = agg * nd_ref[...] + b_ref[...]


def _tc(body, out_shape, *args):
    return pl.pallas_call(body, out_shape=out_shape)(*args)


def kernel(x, edge_index, W0, b0, g0, be0, W1, b1, g1, be1, W2, b2):
    f32 = jnp.float32
    ei = edge_index.reshape(2 * E)
    zeros_d = jnp.zeros((NPAD, D), f32)
    W2p = jnp.pad(W2, ((0, 0), (0, D - NCLS)))

    degp = _deg_kernel(ei, zeros_d)
    xw = _tc(_mm_body, jax.ShapeDtypeStruct((N, D), f32), x, W0)
    h0, ns, nd = _tc(
        _prep_body,
        (jax.ShapeDtypeStruct((N, D), f32),
         jax.ShapeDtypeStruct((N, 1), f32),
         jax.ShapeDtypeStruct((N, 1), f32)),
        xw, degp)

    aggp0 = _mp(h0, ei, zeros_d)
    h1 = _tc(_mid_body, jax.ShapeDtypeStruct((N, D), f32),
             aggp0, nd, b0.reshape(1, D), g0.reshape(1, D),
             be0.reshape(1, D), ns, W1)

    aggp1 = _mp(h1, ei, zeros_d)
    h2 = _tc(_mid_body, jax.ShapeDtypeStruct((N, D), f32),
             aggp1, nd, b1.reshape(1, D), g1.reshape(1, D),
             be1.reshape(1, D), ns, W2p)

    aggp2 = _mp(h2, ei, zeros_d)
    out = _tc(_fin_body, jax.ShapeDtypeStruct((N, NCLS), f32),
              aggp2, nd, b2.reshape(1, NCLS))
    return out
